# baseline (speedup 1.0000x reference)
import functools

import jax
import jax.numpy as jnp
from jax.experimental import pallas as pl
from jax.experimental.pallas import tpu as pltpu

_KB = 1536
_NS = 4


def _mm_kernel(*refs, k_total):
    x_refs = refs[:_NS]
    w_refs = refs[_NS:2 * _NS]
    b_ref = refs[2 * _NS]
    o_ref = refs[2 * _NS + 1]
    k = pl.program_id(0)
    nk = pl.num_programs(0)

    @pl.when(k == 0)
    def _init():
        o_ref[...] = jnp.zeros_like(o_ref)

    def dot_i(i, masked):
        x = x_refs[i][...]
        w = w_refs[i][...]
        if masked:
            valid = k_total - (nk - 1) * _NS * _KB - i * _KB
            cols = jax.lax.broadcasted_iota(jnp.int32, x.shape, 1)
            rows = jax.lax.broadcasted_iota(jnp.int32, w.shape, 0)
            x = jnp.where(cols < valid, x, 0.0)
            w = jnp.where(rows < valid, w, 0.0)
        return jax.lax.dot(
            x.astype(jnp.bfloat16),
            w.astype(jnp.bfloat16),
            preferred_element_type=jnp.float32,
        )

    @pl.when(k < nk - 1)
    def _full():
        acc = dot_i(0, False)
        for i in range(1, _NS):
            acc += dot_i(i, False)
        o_ref[...] += acc

    @pl.when(k == nk - 1)
    def _tail():
        base = (nk - 1) * _NS * _KB
        acc = jnp.zeros_like(o_ref)
        for i in range(_NS):
            valid = k_total - base - i * _KB
            if valid <= 0:
                continue
            acc += dot_i(i, masked=valid < _KB)
        o_ref[...] += acc + b_ref[...]


def kernel(X, W, b):
    B, K = X.shape
    _, N = W.shape
    nk = pl.cdiv(K, _NS * _KB)
    b2 = b.reshape(1, N)
    max_blk = pl.cdiv(K, _KB) - 1

    x_specs = [
        pl.BlockSpec(
            (B, _KB),
            functools.partial(
                lambda i, k: (0, jnp.minimum(_NS * k + i, max_blk)), i
            ),
        )
        for i in range(_NS)
    ]
    w_specs = [
        pl.BlockSpec(
            (_KB, N),
            functools.partial(
                lambda i, k: (jnp.minimum(_NS * k + i, max_blk), 0), i
            ),
        )
        for i in range(_NS)
    ]
    return pl.pallas_call(
        functools.partial(_mm_kernel, k_total=K),
        grid=(nk,),
        in_specs=x_specs + w_specs + [pl.BlockSpec((1, N), lambda k: (0, 0))],
        out_specs=pl.BlockSpec((B, N), lambda k: (0, 0)),
        out_shape=jax.ShapeDtypeStruct((B, N), jnp.float32),
        compiler_params=pltpu.CompilerParams(
            dimension_semantics=("arbitrary",),
        ),
    )(*([X] * _NS), *([W] * _NS), b2)
